# baseline (speedup 1.0000x reference)
import functools

import jax
import jax.numpy as jnp
from jax import lax
from jax.experimental import pallas as pl
from jax.experimental.pallas import tpu as pltpu
from jax.experimental.pallas import tpu_sc as plsc

N_FEAT = 26
B = 1024
L = 50
CARD = 101
EDIM = 8
D = N_FEAT * EDIM
H = 128
G3 = 3 * H
TOP = 32
ROWS = B * L

_NC = 2
_NS = 16
_NW = _NC * _NS
_RW = ROWS // _NW
_NV = _RW // 16
_XG = L * B * 256 // EDIM


def _sc_gather_kernel(table_ref, feat_ref, out_ref, gidx_v, pos_v, rows_v,
                      sem_g, sem_s):
    wid = lax.axis_index("s") * _NC + lax.axis_index("c")
    qbase = wid * _RW
    lanes = lax.iota(jnp.int32, 16)
    q0 = qbase + lanes

    for f in range(N_FEAT):
        pltpu.sync_copy(feat_ref.at[pl.ds(f * ROWS + qbase, _RW)], gidx_v)
        cst = (f // 16) * 128 + (f % 16)

        def vec_body(m, carry, cst=cst, foff=f * CARD):
            sl = pl.ds(m * 16, 16)
            gidx_v[sl] = gidx_v[sl] + foff
            q = q0 + m * 16
            b = q // L
            l = q - b * L
            pos = ((l << 7) + (b >> 3)) * 256 + ((b & 7) << 4) + cst
            pos_v[sl] = pos
            return carry

        lax.fori_loop(0, _NV, vec_body, 0)
        pltpu.async_copy(table_ref.at[gidx_v], rows_v, sem_g).wait()
        pltpu.async_copy(rows_v, out_ref.at[pos_v], sem_s).wait()


def _gru_kernel(xf0_ref, xf1_ref, xb0_ref, xb1_ref,
                wxf0_ref, wxf1_ref, bxf_ref, wxb0_ref, wxb1_ref, bxb_ref,
                whhf_ref, bhhf_ref, whhb_ref, bhhb_ref,
                wc_ref, bc_ref, wh_ref, bh_ref, out_ref,
                hf, hb, mxf, mxb, smf, smb):
    l = pl.program_id(0)

    @pl.when(l == 0)
    def _init():
        zeros = jnp.zeros((B, H), dtype=jnp.float32)
        neg = jnp.full((B, H), -1e30, dtype=jnp.float32)
        hf[...] = zeros
        hb[...] = zeros
        smf[...] = zeros
        smb[...] = zeros
        mxf[...] = neg
        mxb[...] = neg

    pad_mask = jax.lax.broadcasted_iota(jnp.int32, (1, 128), 1) < (D - 128)

    def step(x0_ref, x1_ref, w0_ref, w1_ref, bx_ref, h, whhT_ref, bhh_ref):
        x0 = x0_ref[0, :, 0].reshape(B, 128)
        x1 = jnp.where(pad_mask, x1_ref[0, :, 0].reshape(B, 128), 0.0)
        gi = (jnp.dot(x0, w0_ref[...], preferred_element_type=jnp.float32)
              + jnp.dot(x1, w1_ref[...], preferred_element_type=jnp.float32)
              + bx_ref[0])
        gh = jnp.dot(h, whhT_ref[...], preferred_element_type=jnp.float32) + bhh_ref[0]
        r = jax.nn.sigmoid(gi[:, :H] + gh[:, :H])
        z = jax.nn.sigmoid(gi[:, H:2 * H] + gh[:, H:2 * H])
        n = jnp.tanh(gi[:, 2 * H:] + r * gh[:, 2 * H:])
        return (1.0 - z) * n + z * h

    hf_new = step(xf0_ref, xf1_ref, wxf0_ref, wxf1_ref, bxf_ref,
                  hf[...], whhf_ref, bhhf_ref)
    hb_new = step(xb0_ref, xb1_ref, wxb0_ref, wxb1_ref, bxb_ref,
                  hb[...], whhb_ref, bhhb_ref)
    hf[...] = hf_new
    hb[...] = hb_new
    mxf[...] = jnp.maximum(mxf[...], hf_new)
    mxb[...] = jnp.maximum(mxb[...], hb_new)
    smf[...] = smf[...] + hf_new
    smb[...] = smb[...] + hb_new

    @pl.when(l == L - 1)
    def _head():
        inv_l = 1.0 / L
        combined = jnp.concatenate(
            [hf[...], hb[...], mxf[...], mxb[...], smf[...] * inv_l, smb[...] * inv_l],
            axis=1)
        act = jax.nn.relu(
            jnp.dot(combined, wc_ref[...], preferred_element_type=jnp.float32)
            + bc_ref[0])
        out_ref[...] = jnp.sum(act * wh_ref[0][None, :], axis=1, keepdims=True) + bh_ref[0]


def kernel(features, emb, Wih_f, Whh_f, bih_f, bhh_f, Wih_b, Whh_b, bih_b, bhh_b,
           Wc, bc, Wh, bh):
    feat_flat = features.reshape(N_FEAT * ROWS)
    emb_flat = emb.reshape(N_FEAT * CARD, EDIM)
    zpad = jnp.zeros((256 - D, G3), dtype=jnp.float32)
    Wxf0 = Wih_f.T[:128]
    Wxf1 = jnp.concatenate([Wih_f.T[128:], zpad], axis=0)
    Wxb0 = Wih_b.T[:128]
    Wxb1 = jnp.concatenate([Wih_b.T[128:], zpad], axis=0)
    bxf = bih_f.reshape(1, G3)
    bxb = bih_b.reshape(1, G3)
    WhhfT = Whh_f.T
    WhhbT = Whh_b.T
    bhhf2 = bhh_f.reshape(1, G3)
    bhhb2 = bhh_b.reshape(1, G3)
    WcT = Wc.T
    bc2 = bc.reshape(1, TOP)
    bh2 = bh.reshape(1, 1)

    gather = functools.partial(
        pl.kernel,
        mesh=plsc.VectorSubcoreMesh(core_axis_name="c", subcore_axis_name="s"),
        out_type=jax.ShapeDtypeStruct((_XG, EDIM), jnp.float32),
        scratch_types=[
            pltpu.VMEM((_RW,), jnp.int32),
            pltpu.VMEM((_RW,), jnp.int32),
            pltpu.VMEM((_RW, EDIM), jnp.float32),
            pltpu.SemaphoreType.DMA,
            pltpu.SemaphoreType.DMA,
        ],
        compiler_params=pltpu.CompilerParams(
            needs_layout_passes=False, use_tc_tiling_on_sc=False),
    )(_sc_gather_kernel)
    x = gather(emb_flat, feat_flat)
    x5 = x.reshape(L, B // 8, 2, 8, 128)

    xspec = [
        pl.BlockSpec((1, B // 8, 1, 8, 128), lambda l: (l, 0, 0, 0, 0)),
        pl.BlockSpec((1, B // 8, 1, 8, 128), lambda l: (l, 0, 1, 0, 0)),
        pl.BlockSpec((1, B // 8, 1, 8, 128), lambda l: (L - 1 - l, 0, 0, 0, 0)),
        pl.BlockSpec((1, B // 8, 1, 8, 128), lambda l: (L - 1 - l, 0, 1, 0, 0)),
    ]
    out = pl.pallas_call(
        _gru_kernel,
        grid=(L,),
        in_specs=xspec + [
            pl.BlockSpec((128, G3), lambda l: (0, 0)),
            pl.BlockSpec((128, G3), lambda l: (0, 0)),
            pl.BlockSpec((1, G3), lambda l: (0, 0)),
            pl.BlockSpec((128, G3), lambda l: (0, 0)),
            pl.BlockSpec((128, G3), lambda l: (0, 0)),
            pl.BlockSpec((1, G3), lambda l: (0, 0)),
            pl.BlockSpec((H, G3), lambda l: (0, 0)),
            pl.BlockSpec((1, G3), lambda l: (0, 0)),
            pl.BlockSpec((H, G3), lambda l: (0, 0)),
            pl.BlockSpec((1, G3), lambda l: (0, 0)),
            pl.BlockSpec((6 * H, TOP), lambda l: (0, 0)),
            pl.BlockSpec((1, TOP), lambda l: (0, 0)),
            pl.BlockSpec((1, TOP), lambda l: (0, 0)),
            pl.BlockSpec((1, 1), lambda l: (0, 0)),
        ],
        out_specs=pl.BlockSpec((B, 1), lambda l: (0, 0)),
        out_shape=jax.ShapeDtypeStruct((B, 1), jnp.float32),
        scratch_shapes=[pltpu.VMEM((B, H), jnp.float32)] * 6,
        compiler_params=pltpu.CompilerParams(
            dimension_semantics=("arbitrary",)),
    )(x5, x5, x5, x5, Wxf0, Wxf1, bxf, Wxb0, Wxb1, bxb,
      WhhfT, bhhf2, WhhbT, bhhb2, WcT, bc2, Wh, bh2)
    return out


# speedup vs baseline: 1.4977x; 1.4977x over previous
import functools

import jax
import jax.numpy as jnp
from jax import lax
from jax.experimental import pallas as pl
from jax.experimental.pallas import tpu as pltpu
from jax.experimental.pallas import tpu_sc as plsc

N_FEAT = 26
B = 1024
L = 50
CARD = 101
EDIM = 8
D = N_FEAT * EDIM
H = 128
G3 = 3 * H
TOP = 32
ROWS = B * L

_NC = 2
_NS = 16
_NW = _NC * _NS
_SC_TOTAL = ROWS * N_FEAT
_PER_W = _SC_TOTAL // _NW
_CH = 5200
_NCHUNK = _PER_W // _CH


def _sc_gather_kernel(table_ref, idx_ref, out_ref, idx_v, rows_v,
                      sem_i0, sem_i1, sem_g, sem_o0, sem_o1):
    wid = lax.axis_index("s") * _NC + lax.axis_index("c")
    base = wid * _PER_W
    sem_i = (sem_i0, sem_i1)
    sem_o = (sem_o0, sem_o1)

    def idx_copy(ci, s):
        return pltpu.make_async_copy(
            idx_ref.at[pl.ds(base + ci * _CH, _CH)], idx_v.at[s], sem_i[s])

    def out_copy(ci, s):
        return pltpu.make_async_copy(
            rows_v.at[s], out_ref.at[pl.ds(base + ci * _CH, _CH)], sem_o[s])

    idx_copy(0, 0).start()
    for ci in range(_NCHUNK):
        s = ci % 2
        if ci + 1 < _NCHUNK:
            idx_copy(ci + 1, 1 - s).start()
        idx_copy(ci, s).wait()
        if ci >= 2:
            out_copy(ci - 2, s).wait()
        pltpu.async_copy(table_ref.at[idx_v.at[s]], rows_v.at[s], sem_g).wait()
        out_copy(ci, s).start()
    out_copy(_NCHUNK - 2, _NCHUNK % 2).wait()
    out_copy(_NCHUNK - 1, (_NCHUNK - 1) % 2).wait()


def _gru_kernel(xf_ref, xb_ref, wxf_ref, bxf_ref, wxb_ref, bxb_ref,
                whhf_ref, bhhf_ref, whhb_ref, bhhb_ref,
                wc_ref, bc_ref, wh_ref, bh_ref, out_ref,
                hf, hb, mxf, mxb, smf, smb):
    l = pl.program_id(0)

    @pl.when(l == 0)
    def _init():
        zeros = jnp.zeros((B, H), dtype=jnp.float32)
        neg = jnp.full((B, H), -1e30, dtype=jnp.float32)
        hf[...] = zeros
        hb[...] = zeros
        smf[...] = zeros
        smb[...] = zeros
        mxf[...] = neg
        mxb[...] = neg

    def step(x_blk, wx_ref, bx_ref, h, whhT_ref, bhh_ref):
        gi = jnp.dot(x_blk, wx_ref[...], preferred_element_type=jnp.float32) + bx_ref[0]
        gh = jnp.dot(h, whhT_ref[...], preferred_element_type=jnp.float32) + bhh_ref[0]
        r = jax.nn.sigmoid(gi[:, :H] + gh[:, :H])
        z = jax.nn.sigmoid(gi[:, H:2 * H] + gh[:, H:2 * H])
        n = jnp.tanh(gi[:, 2 * H:] + r * gh[:, 2 * H:])
        return (1.0 - z) * n + z * h

    hf_new = step(xf_ref[0], wxf_ref, bxf_ref, hf[...], whhf_ref, bhhf_ref)
    hb_new = step(xb_ref[0], wxb_ref, bxb_ref, hb[...], whhb_ref, bhhb_ref)
    hf[...] = hf_new
    hb[...] = hb_new
    mxf[...] = jnp.maximum(mxf[...], hf_new)
    mxb[...] = jnp.maximum(mxb[...], hb_new)
    smf[...] = smf[...] + hf_new
    smb[...] = smb[...] + hb_new

    @pl.when(l == L - 1)
    def _head():
        inv_l = 1.0 / L
        combined = jnp.concatenate(
            [hf[...], hb[...], mxf[...], mxb[...], smf[...] * inv_l, smb[...] * inv_l],
            axis=1)
        act = jax.nn.relu(
            jnp.dot(combined, wc_ref[...], preferred_element_type=jnp.float32)
            + bc_ref[0])
        out_ref[...] = jnp.sum(act * wh_ref[0][None, :], axis=1, keepdims=True) + bh_ref[0]


def kernel(features, emb, Wih_f, Whh_f, bih_f, bhh_f, Wih_b, Whh_b, bih_b, bhh_b,
           Wc, bc, Wh, bh):
    feat3 = jnp.transpose(features, (2, 1, 0)).reshape(ROWS, N_FEAT)
    idx_flat = (feat3 + CARD * jnp.arange(N_FEAT, dtype=jnp.int32)[None, :]
                ).reshape(_SC_TOTAL)
    emb_flat = emb.reshape(N_FEAT * CARD, EDIM)
    Wxf = Wih_f.T
    Wxb = Wih_b.T
    bxf = bih_f.reshape(1, G3)
    bxb = bih_b.reshape(1, G3)
    WhhfT = Whh_f.T
    WhhbT = Whh_b.T
    bhhf2 = bhh_f.reshape(1, G3)
    bhhb2 = bhh_b.reshape(1, G3)
    WcT = Wc.T
    bc2 = bc.reshape(1, TOP)
    bh2 = bh.reshape(1, 1)

    gather = functools.partial(
        pl.kernel,
        mesh=plsc.VectorSubcoreMesh(core_axis_name="c", subcore_axis_name="s"),
        out_type=jax.ShapeDtypeStruct((_SC_TOTAL, EDIM), jnp.float32),
        scratch_types=[
            pltpu.VMEM((2, _CH), jnp.int32),
            pltpu.VMEM((2, _CH, EDIM), jnp.float32),
            pltpu.SemaphoreType.DMA,
            pltpu.SemaphoreType.DMA,
            pltpu.SemaphoreType.DMA,
            pltpu.SemaphoreType.DMA,
            pltpu.SemaphoreType.DMA,
        ],
        compiler_params=pltpu.CompilerParams(
            needs_layout_passes=False, use_tc_tiling_on_sc=False),
    )(_sc_gather_kernel)
    x = gather(emb_flat, idx_flat).reshape(L, B, D)

    out = pl.pallas_call(
        _gru_kernel,
        grid=(L,),
        in_specs=[
            pl.BlockSpec((1, B, D), lambda l: (l, 0, 0)),
            pl.BlockSpec((1, B, D), lambda l: (L - 1 - l, 0, 0)),
            pl.BlockSpec((D, G3), lambda l: (0, 0)),
            pl.BlockSpec((1, G3), lambda l: (0, 0)),
            pl.BlockSpec((D, G3), lambda l: (0, 0)),
            pl.BlockSpec((1, G3), lambda l: (0, 0)),
            pl.BlockSpec((H, G3), lambda l: (0, 0)),
            pl.BlockSpec((1, G3), lambda l: (0, 0)),
            pl.BlockSpec((H, G3), lambda l: (0, 0)),
            pl.BlockSpec((1, G3), lambda l: (0, 0)),
            pl.BlockSpec((6 * H, TOP), lambda l: (0, 0)),
            pl.BlockSpec((1, TOP), lambda l: (0, 0)),
            pl.BlockSpec((1, TOP), lambda l: (0, 0)),
            pl.BlockSpec((1, 1), lambda l: (0, 0)),
        ],
        out_specs=pl.BlockSpec((B, 1), lambda l: (0, 0)),
        out_shape=jax.ShapeDtypeStruct((B, 1), jnp.float32),
        scratch_shapes=[pltpu.VMEM((B, H), jnp.float32)] * 6,
        compiler_params=pltpu.CompilerParams(
            dimension_semantics=("arbitrary",)),
    )(x, x, Wxf, bxf, Wxb, bxb, WhhfT, bhhf2, WhhbT, bhhb2, WcT, bc2, Wh, bh2)
    return out
